# baseline (speedup 1.0000x reference)
import functools

import jax
import jax.numpy as jnp
from jax import lax
from jax.experimental import pallas as pl
from jax.experimental.pallas import tpu as pltpu
from jax.experimental.pallas import tpu_sc as plsc

N_NODES = 10000
N_PAD = 10240
N_EDGES = 320000
D_FEAT = 128
HIDDEN = 64

NC = 2
NS = 16
NW = NC * NS
C = 128
NCH = 80
EW = NCH * C
E_PAD = NW * EW
ZR = N_PAD // NS
NBUF = 8
GA = 6
SLAG = 2


def _mesh():
    return plsc.VectorSubcoreMesh(core_axis_name="c", subcore_axis_name="s")


_SC_PARAMS = pltpu.CompilerParams(use_tc_tiling_on_sc=False)


def _sc_degree(dst3):
    @functools.partial(
        pl.kernel,
        mesh=_mesh(),
        out_type=jax.ShapeDtypeStruct((NC, N_PAD), jnp.float32),
        scratch_types=[
            pltpu.VMEM((NCH, C), jnp.int32),
            pltpu.VMEM((C,), jnp.float32),
            pltpu.VMEM((ZR,), jnp.float32),
            pltpu.VMEM_SHARED((N_PAD,), jnp.float32),
            pltpu.SemaphoreType.DMA,
        ],
        compiler_params=_SC_PARAMS,
    )
    def k(dst_h, out_h, dst_v, ones_v, zbuf, acc, sem):
        c = lax.axis_index("c")
        s = lax.axis_index("s")
        wid = s * NC + c
        pltpu.sync_copy(dst_h.at[wid], dst_v)
        for i in range(C // 16):
            ones_v[pl.ds(i * 16, 16)] = jnp.ones((16,), jnp.float32)

        def zfill(i, carry):
            zbuf[pl.ds(i * 16, 16)] = jnp.zeros((16,), jnp.float32)
            return carry

        lax.fori_loop(0, ZR // 16, zfill, 0)
        pltpu.sync_copy(zbuf, acc.at[pl.ds(s * ZR, ZR)])
        plsc.subcore_barrier()

        def fire(j, carry):
            pltpu.async_copy(ones_v, acc.at[dst_v.at[j]], sem, add=True)
            return carry

        lax.fori_loop(0, NCH, fire, 0)

        def drain(j, carry):
            pltpu.make_async_copy(ones_v, acc.at[dst_v.at[0]], sem).wait()
            return carry

        lax.fori_loop(0, NCH, drain, 0)
        plsc.subcore_barrier()
        pltpu.sync_copy(acc.at[pl.ds(s * ZR, ZR)], out_h.at[c, pl.ds(s * ZR, ZR)])

    return k(dst3)


def _sc_gather_scatter(table, src3, dst3, feat, stage_table, init_from_table):
    wide = feat > 1
    buf_t = pltpu.VMEM((NBUF, C, feat) if wide else (NBUF, C), jnp.float32)
    acc_t = pltpu.VMEM_SHARED((N_PAD, feat) if wide else (N_PAD,), jnp.float32)
    out_sh = (NC, N_PAD, feat) if wide else (NC, N_PAD)
    scratch = [
        pltpu.VMEM((NCH, C), jnp.int32),
        pltpu.VMEM((NCH, C), jnp.int32),
        buf_t,
        acc_t,
    ]
    if stage_table:
        scratch.append(pltpu.VMEM_SHARED((N_PAD, feat) if wide else (N_PAD,),
                                         jnp.float32))
    scratch += [pltpu.SemaphoreType.DMA] * (2 * NBUF)

    @functools.partial(
        pl.kernel,
        mesh=_mesh(),
        out_type=jax.ShapeDtypeStruct(out_sh, jnp.float32),
        scratch_types=scratch,
        compiler_params=_SC_PARAMS,
    )
    def k(tab_h, src_h, dst_h, out_h, src_v, dst_v, bufs, *rest):
        if stage_table:
            tab = rest[1]
            acc = rest[0]
            sems = rest[2:]
        else:
            tab = tab_h
            acc = rest[0]
            sems = rest[1:]
        gsem = sems[:NBUF]
        ssem = sems[NBUF:]
        c = lax.axis_index("c")
        s = lax.axis_index("s")
        wid = s * NC + c
        pltpu.sync_copy(src_h.at[wid], src_v)
        pltpu.sync_copy(dst_h.at[wid], dst_v)
        if wide:
            def zfill(i, carry):
                for j in range(feat // 16):
                    bufs[0, i, pl.ds(j * 16, 16)] = jnp.zeros((16,),
                                                              jnp.float32)
                return carry
        else:
            def zfill(i, carry):
                bufs[0, pl.ds(i * 16, 16)] = jnp.zeros((16,), jnp.float32)
                return carry

        lax.fori_loop(0, C if wide else C // 16, zfill, 0)
        if init_from_table:
            @pl.when(c == 0)
            def _():
                pltpu.sync_copy(tab_h.at[pl.ds(s * ZR, ZR)],
                                acc.at[pl.ds(s * ZR, ZR)])

            @pl.when(c != 0)
            def _():
                for kk in range(ZR // C):
                    pltpu.sync_copy(bufs.at[0],
                                    acc.at[pl.ds(s * ZR + kk * C, C)])
        else:
            for kk in range(ZR // C):
                pltpu.sync_copy(bufs.at[0], acc.at[pl.ds(s * ZR + kk * C, C)])
        if stage_table:
            pltpu.sync_copy(tab_h.at[pl.ds(s * ZR, ZR)],
                            tab.at[pl.ds(s * ZR, ZR)])
        plsc.subcore_barrier()

        def gfire(j, b):
            pltpu.async_copy(tab.at[src_v.at[j]], bufs.at[b], gsem[b])

        def gwait(j, b):
            pltpu.make_async_copy(tab.at[src_v.at[j]], bufs.at[b],
                                  gsem[b]).wait()

        def sfire(j, b):
            pltpu.async_copy(bufs.at[b], acc.at[dst_v.at[j]], ssem[b],
                             add=True)

        def sdrain(j, b):
            pltpu.make_async_copy(bufs.at[b], acc.at[dst_v.at[j]],
                                  ssem[b]).wait()

        for j in range(GA):
            gfire(j, j)
        for b in range(NBUF):
            gwait(b, b)
            sfire(b, b)
            if b >= SLAG:
                sdrain(b - SLAG, (b - SLAG) % NBUF)
            gfire(b + GA, (b + GA) % NBUF)

        def body(t, carry):
            for b in range(NBUF):
                j = t * NBUF + b
                gwait(j, b)
                sfire(j, b)
                sdrain(j - SLAG, (b - SLAG) % NBUF)
                gfire(j + GA, (b + GA) % NBUF)
            return carry

        lax.fori_loop(1, (NCH // NBUF) - 1, body, 0)
        t = NCH // NBUF - 1
        for b in range(NBUF):
            j = t * NBUF + b
            gwait(j, b)
            sfire(j, b)
            sdrain(j - SLAG, (b - SLAG) % NBUF)
            if j + GA < NCH:
                gfire(j + GA, (b + GA) % NBUF)
        for b in range(NBUF - SLAG, NBUF):
            sdrain(t * NBUF + b, b)
        plsc.subcore_barrier()
        if wide:
            pltpu.sync_copy(acc.at[pl.ds(s * ZR, ZR)],
                            out_h.at[c, pl.ds(s * ZR, ZR)])
        else:
            pltpu.sync_copy(acc.at[pl.ds(s * ZR, ZR)],
                            out_h.at[c, pl.ds(s * ZR, ZR)])

    return k(table, src3, dst3)


_RB = 2048
_SB = _RB // 128


def _tc1a(x_p, W1):
    def body(x_ref, w_ref, h_ref):
        h_ref[...] = jnp.dot(x_ref[...], w_ref[...],
                             preferred_element_type=jnp.float32)

    grid = (N_PAD // _RB,)
    return pl.pallas_call(
        body,
        grid=grid,
        in_specs=[
            pl.BlockSpec((_RB, D_FEAT), lambda i: (i, 0)),
            pl.BlockSpec((D_FEAT, HIDDEN), lambda i: (0, 0)),
        ],
        out_specs=pl.BlockSpec((_RB, HIDDEN), lambda i: (i, 0)),
        out_shape=jax.ShapeDtypeStruct((N_PAD, HIDDEN), jnp.float32),
    )(x_p, W1)


def _tc1b(h, d0, d1):
    def body(h_ref, d0_ref, d1_ref, hp_ref, isd_ref):
        deg = d0_ref[...] + d1_ref[...] + 1.0
        isd = lax.rsqrt(deg)
        h3 = h_ref[...].reshape(_SB, 128, HIDDEN)
        hp_ref[...] = (h3 * isd.reshape(_SB, 128, 1)).reshape(_RB, HIDDEN)
        isd_ref[...] = isd

    grid = (N_PAD // _RB,)
    return pl.pallas_call(
        body,
        grid=grid,
        in_specs=[
            pl.BlockSpec((_RB, HIDDEN), lambda i: (i, 0)),
            pl.BlockSpec((_SB, 128), lambda i: (i, 0)),
            pl.BlockSpec((_SB, 128), lambda i: (i, 0)),
        ],
        out_specs=[
            pl.BlockSpec((_RB, HIDDEN), lambda i: (i, 0)),
            pl.BlockSpec((_SB, 128), lambda i: (i, 0)),
        ],
        out_shape=[
            jax.ShapeDtypeStruct((N_PAD, HIDDEN), jnp.float32),
            jax.ShapeDtypeStruct((N_PAD // 128, 128), jnp.float32),
        ],
    )(h, d0, d1)


def _tc2(p0, p1, isd_rep, isd_pair, w2p, b1t):
    def body(p0_ref, p1_ref, ir_ref, ip_ref, w2_ref, b1_ref, sp_ref):
        pre = (p0_ref[...] + p1_ref[...]) * ir_ref[...] + b1_ref[...]
        r = jnp.maximum(pre, 0.0)
        s = jnp.dot(r, w2_ref[...], preferred_element_type=jnp.float32)
        sp_ref[...] = s * ip_ref[...]

    pb = _RB // 2
    grid = (N_PAD // _RB,)
    return pl.pallas_call(
        body,
        grid=grid,
        in_specs=[
            pl.BlockSpec((pb, 128), lambda i: (i, 0)),
            pl.BlockSpec((pb, 128), lambda i: (i, 0)),
            pl.BlockSpec((pb, 128), lambda i: (i, 0)),
            pl.BlockSpec((pb, 2), lambda i: (i, 0)),
            pl.BlockSpec((128, 2), lambda i: (0, 0)),
            pl.BlockSpec((1, 128), lambda i: (0, 0)),
        ],
        out_specs=pl.BlockSpec((pb, 2), lambda i: (i, 0)),
        out_shape=jax.ShapeDtypeStruct((N_PAD // 2, 2), jnp.float32),
    )(p0, p1, isd_rep, isd_pair, w2p, b1t)


def _tc3(q0, q1, isd, b2):
    def body(q0_ref, q1_ref, isd_ref, b2_ref, out_ref):
        out_ref[...] = (isd_ref[...] * (q0_ref[...] + q1_ref[...])
                        + b2_ref[0, 0])

    nb = N_PAD // 128
    return pl.pallas_call(
        body,
        in_specs=[
            pl.BlockSpec((nb, 128), lambda: (0, 0)),
            pl.BlockSpec((nb, 128), lambda: (0, 0)),
            pl.BlockSpec((nb, 128), lambda: (0, 0)),
            pl.BlockSpec((1, 1), lambda: (0, 0)),
        ],
        out_specs=pl.BlockSpec((nb, 128), lambda: (0, 0)),
        out_shape=jax.ShapeDtypeStruct((nb, 128), jnp.float32),
    )(q0, q1, isd, b2)


def kernel(x, edge_index, W1, b1, W2, b2):
    ei_p = jnp.pad(edge_index, ((0, 0), (0, E_PAD - N_EDGES)))
    col = lax.broadcasted_iota(jnp.int32, (2, E_PAD), 1)
    ei_p = jnp.where(col < N_EDGES, ei_p,
                     N_NODES + col % (N_PAD - N_NODES))
    src3 = ei_p[0].reshape(NW, NCH, C)
    dst3 = ei_p[1].reshape(NW, NCH, C)

    h = _tc1a(x, W1)
    dp = _sc_degree(dst3)
    d0 = dp[0].reshape(N_PAD // 128, 128)
    d1 = dp[1].reshape(N_PAD // 128, 128)
    hp, isd = _tc1b(h, d0, d1)
    pp = _sc_gather_scatter(hp, src3, dst3, HIDDEN, False, True)
    isd_pair = isd.reshape(N_PAD // 2, 2)
    isd_rep = jnp.repeat(isd_pair, HIDDEN, axis=1)
    w2col = W2[:, 0]
    w2p = (jnp.zeros((128, 2), jnp.float32)
           .at[:HIDDEN, 0].set(w2col).at[HIDDEN:, 1].set(w2col))
    b1t = jnp.concatenate([b1, b1]).reshape(1, 128)
    sp = _tc2(pp[0].reshape(N_PAD // 2, 128), pp[1].reshape(N_PAD // 2, 128),
              isd_rep, isd_pair, w2p, b1t)
    qp = _sc_gather_scatter(sp.reshape(N_PAD), src3, dst3, 1, True, True)
    out = _tc3(qp[0].reshape(N_PAD // 128, 128),
               qp[1].reshape(N_PAD // 128, 128), isd, b2.reshape(1, 1))
    return out.reshape(-1)[:N_NODES]


# speedup vs baseline: 1.0601x; 1.0601x over previous
import functools

import jax
import jax.numpy as jnp
from jax import lax
from jax.experimental import pallas as pl
from jax.experimental.pallas import tpu as pltpu
from jax.experimental.pallas import tpu_sc as plsc

N_NODES = 10000
N_PAD = 10240
N_EDGES = 320000
D_FEAT = 128
HIDDEN = 64

NC = 2
NS = 16
NW = NC * NS
C = 128
NCH = 80
EW = NCH * C
E_PAD = NW * EW
ZR = N_PAD // NS
NBUF = 8
GA = 6
SLAG = 2


def _mesh():
    return plsc.VectorSubcoreMesh(core_axis_name="c", subcore_axis_name="s")


_SC_PARAMS = pltpu.CompilerParams(use_tc_tiling_on_sc=False)


def _sc_degree(dst3):
    @functools.partial(
        pl.kernel,
        mesh=_mesh(),
        out_type=jax.ShapeDtypeStruct((NC, N_PAD), jnp.float32),
        scratch_types=[
            pltpu.VMEM((NCH, C), jnp.int32),
            pltpu.VMEM((C,), jnp.float32),
            pltpu.VMEM((ZR,), jnp.float32),
            pltpu.VMEM_SHARED((N_PAD,), jnp.float32),
            pltpu.SemaphoreType.DMA,
        ],
        compiler_params=_SC_PARAMS,
    )
    def k(dst_h, out_h, dst_v, ones_v, zbuf, acc, sem):
        c = lax.axis_index("c")
        s = lax.axis_index("s")
        wid = s * NC + c
        pltpu.sync_copy(dst_h.at[wid], dst_v)
        for i in range(C // 16):
            ones_v[pl.ds(i * 16, 16)] = jnp.ones((16,), jnp.float32)

        def zfill(i, carry):
            zbuf[pl.ds(i * 16, 16)] = jnp.zeros((16,), jnp.float32)
            return carry

        lax.fori_loop(0, ZR // 16, zfill, 0)
        pltpu.sync_copy(zbuf, acc.at[pl.ds(s * ZR, ZR)])
        plsc.subcore_barrier()

        def fire(j, carry):
            pltpu.async_copy(ones_v, acc.at[dst_v.at[j]], sem, add=True)
            return carry

        lax.fori_loop(0, NCH, fire, 0)

        def drain(j, carry):
            pltpu.make_async_copy(ones_v, acc.at[dst_v.at[0]], sem).wait()
            return carry

        lax.fori_loop(0, NCH, drain, 0)
        plsc.subcore_barrier()
        pltpu.sync_copy(acc.at[pl.ds(s * ZR, ZR)], out_h.at[c, pl.ds(s * ZR, ZR)])

    return k(dst3)


def _sc_gather_scatter(table, src3, dst3, feat, stage_table, init_from_table):
    wide = feat > 1
    buf_t = pltpu.VMEM((NBUF, C, feat) if wide else (NBUF, C), jnp.float32)
    acc_t = pltpu.VMEM_SHARED((N_PAD, feat) if wide else (N_PAD,), jnp.float32)
    out_sh = (NC, N_PAD, feat) if wide else (NC, N_PAD)
    scratch = [
        pltpu.VMEM((NCH, C), jnp.int32),
        pltpu.VMEM((NCH, C), jnp.int32),
        buf_t,
        acc_t,
    ]
    if stage_table:
        scratch.append(pltpu.VMEM_SHARED((N_PAD, feat) if wide else (N_PAD,),
                                         jnp.float32))
    scratch += [pltpu.SemaphoreType.DMA] * (2 * NBUF)

    @functools.partial(
        pl.kernel,
        mesh=_mesh(),
        out_type=jax.ShapeDtypeStruct(out_sh, jnp.float32),
        scratch_types=scratch,
        compiler_params=_SC_PARAMS,
    )
    def k(tab_h, src_h, dst_h, out_h, src_v, dst_v, bufs, *rest):
        if stage_table:
            tab = rest[1]
            acc = rest[0]
            sems = rest[2:]
        else:
            tab = tab_h
            acc = rest[0]
            sems = rest[1:]
        gsem = sems[:NBUF]
        ssem = sems[NBUF:]
        c = lax.axis_index("c")
        s = lax.axis_index("s")
        wid = s * NC + c
        pltpu.sync_copy(src_h.at[wid], src_v)
        pltpu.sync_copy(dst_h.at[wid], dst_v)
        if wide:
            def zfill(i, carry):
                for j in range(feat // 16):
                    bufs[0, i, pl.ds(j * 16, 16)] = jnp.zeros((16,),
                                                              jnp.float32)
                return carry
        else:
            def zfill(i, carry):
                bufs[0, pl.ds(i * 16, 16)] = jnp.zeros((16,), jnp.float32)
                return carry

        lax.fori_loop(0, C if wide else C // 16, zfill, 0)
        if init_from_table:
            @pl.when(c == 0)
            def _():
                pltpu.sync_copy(tab_h.at[pl.ds(s * ZR, ZR)],
                                acc.at[pl.ds(s * ZR, ZR)])

            @pl.when(c != 0)
            def _():
                for kk in range(ZR // C):
                    pltpu.sync_copy(bufs.at[0],
                                    acc.at[pl.ds(s * ZR + kk * C, C)])
        else:
            for kk in range(ZR // C):
                pltpu.sync_copy(bufs.at[0], acc.at[pl.ds(s * ZR + kk * C, C)])
        if stage_table:
            pltpu.sync_copy(tab_h.at[pl.ds(s * ZR, ZR)],
                            tab.at[pl.ds(s * ZR, ZR)])
        plsc.subcore_barrier()

        def gfire(j, b):
            pltpu.async_copy(tab.at[src_v.at[j]], bufs.at[b], gsem[b])

        def gwait(j, b):
            pltpu.make_async_copy(tab.at[src_v.at[j]], bufs.at[b],
                                  gsem[b]).wait()

        def sfire(j, b):
            pltpu.async_copy(bufs.at[b], acc.at[dst_v.at[j]], ssem[b],
                             add=True)

        def sdrain(j, b):
            pltpu.make_async_copy(bufs.at[b], acc.at[dst_v.at[j]],
                                  ssem[b]).wait()

        for j in range(GA):
            gfire(j, j)
        for b in range(NBUF):
            gwait(b, b)
            sfire(b, b)
            if b >= SLAG:
                sdrain(b - SLAG, (b - SLAG) % NBUF)
            gfire(b + GA, (b + GA) % NBUF)

        def body(t, carry):
            for b in range(NBUF):
                j = t * NBUF + b
                gwait(j, b)
                sfire(j, b)
                sdrain(j - SLAG, (b - SLAG) % NBUF)
                gfire(j + GA, (b + GA) % NBUF)
            return carry

        lax.fori_loop(1, (NCH // NBUF) - 1, body, 0)
        t = NCH // NBUF - 1
        for b in range(NBUF):
            j = t * NBUF + b
            gwait(j, b)
            sfire(j, b)
            sdrain(j - SLAG, (b - SLAG) % NBUF)
            if j + GA < NCH:
                gfire(j + GA, (b + GA) % NBUF)
        for b in range(NBUF - SLAG, NBUF):
            sdrain(t * NBUF + b, b)
        plsc.subcore_barrier()
        pltpu.sync_copy(acc.at[pl.ds(s * ZR, ZR)],
                        out_h.at[c, pl.ds(s * ZR, ZR)])

    return k(table, src3, dst3)


_RB = 2048
_SB = _RB // 128


def _tc1a(x_p, W1):
    def body(x_ref, w_ref, h_ref):
        h_ref[...] = jnp.dot(x_ref[...], w_ref[...],
                             preferred_element_type=jnp.float32)

    grid = (N_PAD // _RB,)
    return pl.pallas_call(
        body,
        grid=grid,
        in_specs=[
            pl.BlockSpec((_RB, D_FEAT), lambda i: (i, 0)),
            pl.BlockSpec((D_FEAT, HIDDEN), lambda i: (0, 0)),
        ],
        out_specs=pl.BlockSpec((_RB, HIDDEN), lambda i: (i, 0)),
        out_shape=jax.ShapeDtypeStruct((N_PAD, HIDDEN), jnp.float32),
    )(x_p, W1)


def _tc1b(h, d0, d1):
    def body(h_ref, d0_ref, d1_ref, hp_ref, isd_ref):
        deg = d0_ref[...] + d1_ref[...] + 1.0
        isd = lax.rsqrt(deg)
        h3 = h_ref[...].reshape(_SB, 128, HIDDEN)
        hp_ref[...] = (h3 * isd.reshape(_SB, 128, 1)).reshape(_RB, HIDDEN)
        isd_ref[...] = isd

    grid = (N_PAD // _RB,)
    return pl.pallas_call(
        body,
        grid=grid,
        in_specs=[
            pl.BlockSpec((_RB, HIDDEN), lambda i: (i, 0)),
            pl.BlockSpec((_SB, 128), lambda i: (i, 0)),
            pl.BlockSpec((_SB, 128), lambda i: (i, 0)),
        ],
        out_specs=[
            pl.BlockSpec((_RB, HIDDEN), lambda i: (i, 0)),
            pl.BlockSpec((_SB, 128), lambda i: (i, 0)),
        ],
        out_shape=[
            jax.ShapeDtypeStruct((N_PAD, HIDDEN), jnp.float32),
            jax.ShapeDtypeStruct((N_PAD // 128, 128), jnp.float32),
        ],
    )(h, d0, d1)


def _tc2(p0, p1, isd, w2row, b1row):
    def body(p0_ref, p1_ref, isd_ref, w2_ref, b1_ref, sp_ref):
        isd3 = isd_ref[...].reshape(_SB, 128, 1)
        pre3 = ((p0_ref[...] + p1_ref[...]).reshape(_SB, 128, HIDDEN) * isd3
                + b1_ref[...].reshape(1, 1, HIDDEN))
        r3 = jnp.maximum(pre3, 0.0)
        s3 = jnp.sum(r3 * w2_ref[...].reshape(1, 1, HIDDEN), axis=-1)
        sp_ref[...] = s3 * isd_ref[...]

    grid = (N_PAD // _RB,)
    return pl.pallas_call(
        body,
        grid=grid,
        in_specs=[
            pl.BlockSpec((_RB, HIDDEN), lambda i: (i, 0)),
            pl.BlockSpec((_RB, HIDDEN), lambda i: (i, 0)),
            pl.BlockSpec((_SB, 128), lambda i: (i, 0)),
            pl.BlockSpec((1, HIDDEN), lambda i: (0, 0)),
            pl.BlockSpec((1, HIDDEN), lambda i: (0, 0)),
        ],
        out_specs=pl.BlockSpec((_SB, 128), lambda i: (i, 0)),
        out_shape=jax.ShapeDtypeStruct((N_PAD // 128, 128), jnp.float32),
    )(p0, p1, isd, w2row, b1row)


def _tc3(q0, q1, isd, b2):
    def body(q0_ref, q1_ref, isd_ref, b2_ref, out_ref):
        out_ref[...] = (isd_ref[...] * (q0_ref[...] + q1_ref[...])
                        + b2_ref[0, 0])

    nb = N_PAD // 128
    return pl.pallas_call(
        body,
        in_specs=[
            pl.BlockSpec((nb, 128), lambda: (0, 0)),
            pl.BlockSpec((nb, 128), lambda: (0, 0)),
            pl.BlockSpec((nb, 128), lambda: (0, 0)),
            pl.BlockSpec((1, 1), lambda: (0, 0)),
        ],
        out_specs=pl.BlockSpec((nb, 128), lambda: (0, 0)),
        out_shape=jax.ShapeDtypeStruct((nb, 128), jnp.float32),
    )(q0, q1, isd, b2)


def kernel(x, edge_index, W1, b1, W2, b2):
    ei_p = jnp.pad(edge_index, ((0, 0), (0, E_PAD - N_EDGES)))
    col = lax.broadcasted_iota(jnp.int32, (2, E_PAD), 1)
    ei_p = jnp.where(col < N_EDGES, ei_p,
                     N_NODES + col % (N_PAD - N_NODES))
    src3 = ei_p[0].reshape(NW, NCH, C)
    dst3 = ei_p[1].reshape(NW, NCH, C)

    h = _tc1a(x, W1)
    dp = _sc_degree(dst3)
    d0 = dp[0].reshape(N_PAD // 128, 128)
    d1 = dp[1].reshape(N_PAD // 128, 128)
    hp, isd = _tc1b(h, d0, d1)
    pp = _sc_gather_scatter(hp, src3, dst3, HIDDEN, False, True)
    sp = _tc2(pp[0], pp[1], isd, W2.reshape(1, HIDDEN),
              b1.reshape(1, HIDDEN))
    qp = _sc_gather_scatter(sp.reshape(N_PAD), src3, dst3, 1, True, True)
    out = _tc3(qp[0].reshape(N_PAD // 128, 128),
               qp[1].reshape(N_PAD // 128, 128), isd, b2.reshape(1, 1))
    return out.reshape(-1)[:N_NODES]


# speedup vs baseline: 1.0624x; 1.0021x over previous
import functools

import jax
import jax.numpy as jnp
from jax import lax
from jax.experimental import pallas as pl
from jax.experimental.pallas import tpu as pltpu
from jax.experimental.pallas import tpu_sc as plsc

N_NODES = 10000
N_PAD = 10240
N_EDGES = 320000
D_FEAT = 128
HIDDEN = 64

NC = 2
NS = 16
NW = NC * NS
C = 128
NCH = 80
EW = NCH * C
E_PAD = NW * EW
ZR = N_PAD // NS
NBUF = 8
GA = 6
SLAG = 2


def _mesh():
    return plsc.VectorSubcoreMesh(core_axis_name="c", subcore_axis_name="s")


_SC_PARAMS = pltpu.CompilerParams(use_tc_tiling_on_sc=False)


def _sc_degree(dst3):
    @functools.partial(
        pl.kernel,
        mesh=_mesh(),
        out_type=jax.ShapeDtypeStruct((NC, N_PAD), jnp.float32),
        scratch_types=[
            pltpu.VMEM((NCH, C), jnp.int32),
            pltpu.VMEM((C,), jnp.float32),
            pltpu.VMEM((ZR,), jnp.float32),
            pltpu.VMEM_SHARED((N_PAD,), jnp.float32),
            pltpu.SemaphoreType.DMA,
        ],
        compiler_params=_SC_PARAMS,
    )
    def k(dst_h, out_h, dst_v, ones_v, zbuf, acc, sem):
        c = lax.axis_index("c")
        s = lax.axis_index("s")
        wid = s * NC + c
        pltpu.sync_copy(dst_h.at[wid], dst_v)
        for i in range(C // 16):
            ones_v[pl.ds(i * 16, 16)] = jnp.ones((16,), jnp.float32)

        def zfill(i, carry):
            zbuf[pl.ds(i * 16, 16)] = jnp.zeros((16,), jnp.float32)
            return carry

        lax.fori_loop(0, ZR // 16, zfill, 0)
        pltpu.sync_copy(zbuf, acc.at[pl.ds(s * ZR, ZR)])
        plsc.subcore_barrier()

        def fire(j, carry):
            pltpu.async_copy(ones_v, acc.at[dst_v.at[j]], sem, add=True)
            return carry

        lax.fori_loop(0, NCH, fire, 0)

        def drain(j, carry):
            pltpu.make_async_copy(ones_v, acc.at[dst_v.at[0]], sem).wait()
            return carry

        lax.fori_loop(0, NCH, drain, 0)
        plsc.subcore_barrier()
        pltpu.sync_copy(acc.at[pl.ds(s * ZR, ZR)], out_h.at[c, pl.ds(s * ZR, ZR)])

    return k(dst3)


def _sc_gather_scatter(table, src3, dst3, feat, stage_table, init_from_table):
    wide = feat > 1
    buf_t = pltpu.VMEM((NBUF, C, feat) if wide else (NBUF, C), jnp.float32)
    acc_t = pltpu.VMEM_SHARED((N_PAD, feat) if wide else (N_PAD,), jnp.float32)
    out_sh = (NC, N_PAD, feat) if wide else (NC, N_PAD)
    scratch = [
        pltpu.VMEM((NCH, C), jnp.int32),
        pltpu.VMEM((NCH, C), jnp.int32),
        buf_t,
        acc_t,
    ]
    if stage_table:
        scratch.append(pltpu.VMEM_SHARED((N_PAD, feat) if wide else (N_PAD,),
                                         jnp.float32))
    scratch += [pltpu.SemaphoreType.DMA] * (2 * NBUF)

    @functools.partial(
        pl.kernel,
        mesh=_mesh(),
        out_type=jax.ShapeDtypeStruct(out_sh, jnp.float32),
        scratch_types=scratch,
        compiler_params=_SC_PARAMS,
    )
    def k(tab_h, src_h, dst_h, out_h, src_v, dst_v, bufs, *rest):
        if stage_table:
            tab = rest[1]
            acc = rest[0]
            sems = rest[2:]
        else:
            tab = tab_h
            acc = rest[0]
            sems = rest[1:]
        gsem = sems[:NBUF]
        ssem = sems[NBUF:]
        c = lax.axis_index("c")
        s = lax.axis_index("s")
        wid = s * NC + c
        pltpu.sync_copy(src_h.at[wid], src_v)
        pltpu.sync_copy(dst_h.at[wid], dst_v)
        if wide:
            def zfill(i, carry):
                for j in range(feat // 16):
                    bufs[0, i, pl.ds(j * 16, 16)] = jnp.zeros((16,),
                                                              jnp.float32)
                return carry
        else:
            def zfill(i, carry):
                bufs[0, pl.ds(i * 16, 16)] = jnp.zeros((16,), jnp.float32)
                return carry

        lax.fori_loop(0, C if wide else C // 16, zfill, 0)
        if init_from_table:
            @pl.when(c == 0)
            def _():
                pltpu.sync_copy(tab_h.at[pl.ds(s * ZR, ZR)],
                                acc.at[pl.ds(s * ZR, ZR)])

            @pl.when(c != 0)
            def _():
                for kk in range(ZR // C):
                    pltpu.sync_copy(bufs.at[0],
                                    acc.at[pl.ds(s * ZR + kk * C, C)])
        else:
            for kk in range(ZR // C):
                pltpu.sync_copy(bufs.at[0], acc.at[pl.ds(s * ZR + kk * C, C)])
        if stage_table:
            pltpu.sync_copy(tab_h.at[pl.ds(s * ZR, ZR)],
                            tab.at[pl.ds(s * ZR, ZR)])
        plsc.subcore_barrier()

        def gfire(j, b):
            pltpu.async_copy(tab.at[src_v.at[j]], bufs.at[b], gsem[b])

        def gwait(j, b):
            pltpu.make_async_copy(tab.at[src_v.at[j]], bufs.at[b],
                                  gsem[b]).wait()

        def sfire(j, b):
            pltpu.async_copy(bufs.at[b], acc.at[dst_v.at[j]], ssem[b],
                             add=True)

        def sdrain(j, b):
            pltpu.make_async_copy(bufs.at[b], acc.at[dst_v.at[j]],
                                  ssem[b]).wait()

        for j in range(GA):
            gfire(j, j)
        for b in range(NBUF):
            gwait(b, b)
            sfire(b, b)
            if b >= SLAG:
                sdrain(b - SLAG, (b - SLAG) % NBUF)
            gfire(b + GA, (b + GA) % NBUF)

        def body(t, carry):
            for b in range(NBUF):
                j = t * NBUF + b
                gwait(j, b)
                sfire(j, b)
                sdrain(j - SLAG, (b - SLAG) % NBUF)
                gfire(j + GA, (b + GA) % NBUF)
            return carry

        lax.fori_loop(1, (NCH // NBUF) - 1, body, 0)
        t = NCH // NBUF - 1
        for b in range(NBUF):
            j = t * NBUF + b
            gwait(j, b)
            sfire(j, b)
            sdrain(j - SLAG, (b - SLAG) % NBUF)
            if j + GA < NCH:
                gfire(j + GA, (b + GA) % NBUF)
        for b in range(NBUF - SLAG, NBUF):
            sdrain(t * NBUF + b, b)
        plsc.subcore_barrier()
        pltpu.sync_copy(acc.at[pl.ds(s * ZR, ZR)],
                        out_h.at[c, pl.ds(s * ZR, ZR)])

    return k(table, src3, dst3)


_RB = 2048
_SB = _RB // 128


def _tc1a(x_p, W1):
    def body(x_ref, w_ref, h_ref):
        h_ref[...] = jnp.dot(x_ref[...].astype(jnp.bfloat16),
                             w_ref[...].astype(jnp.bfloat16),
                             preferred_element_type=jnp.float32)

    grid = (N_PAD // _RB,)
    return pl.pallas_call(
        body,
        grid=grid,
        in_specs=[
            pl.BlockSpec((_RB, D_FEAT), lambda i: (i, 0)),
            pl.BlockSpec((D_FEAT, HIDDEN), lambda i: (0, 0)),
        ],
        out_specs=pl.BlockSpec((_RB, HIDDEN), lambda i: (i, 0)),
        out_shape=jax.ShapeDtypeStruct((N_PAD, HIDDEN), jnp.float32),
    )(x_p, W1)


def _tc1b(h, d0, d1):
    def body(h_ref, d0_ref, d1_ref, hp_ref, isd_ref):
        deg = d0_ref[...] + d1_ref[...] + 1.0
        isd = lax.rsqrt(deg)
        h3 = h_ref[...].reshape(_SB, 128, HIDDEN)
        hp_ref[...] = (h3 * isd.reshape(_SB, 128, 1)).reshape(_RB, HIDDEN)
        isd_ref[...] = isd

    grid = (N_PAD // _RB,)
    return pl.pallas_call(
        body,
        grid=grid,
        in_specs=[
            pl.BlockSpec((_RB, HIDDEN), lambda i: (i, 0)),
            pl.BlockSpec((_SB, 128), lambda i: (i, 0)),
            pl.BlockSpec((_SB, 128), lambda i: (i, 0)),
        ],
        out_specs=[
            pl.BlockSpec((_RB, HIDDEN), lambda i: (i, 0)),
            pl.BlockSpec((_SB, 128), lambda i: (i, 0)),
        ],
        out_shape=[
            jax.ShapeDtypeStruct((N_PAD, HIDDEN), jnp.float32),
            jax.ShapeDtypeStruct((N_PAD // 128, 128), jnp.float32),
        ],
    )(h, d0, d1)


def _tc2(p0, p1, isd, w2row, b1row):
    def body(p0_ref, p1_ref, isd_ref, w2_ref, b1_ref, sp_ref):
        isd3 = isd_ref[...].reshape(_SB, 128, 1)
        pre3 = ((p0_ref[...] + p1_ref[...]).reshape(_SB, 128, HIDDEN) * isd3
                + b1_ref[...].reshape(1, 1, HIDDEN))
        r3 = jnp.maximum(pre3, 0.0)
        r3b = r3.astype(jnp.bfloat16).astype(jnp.float32)
        w2b = (w2_ref[...].astype(jnp.bfloat16).astype(jnp.float32)
               .reshape(1, 1, HIDDEN))
        s3 = jnp.sum(r3b * w2b, axis=-1)
        sp_ref[...] = s3 * isd_ref[...]

    grid = (N_PAD // _RB,)
    return pl.pallas_call(
        body,
        grid=grid,
        in_specs=[
            pl.BlockSpec((_RB, HIDDEN), lambda i: (i, 0)),
            pl.BlockSpec((_RB, HIDDEN), lambda i: (i, 0)),
            pl.BlockSpec((_SB, 128), lambda i: (i, 0)),
            pl.BlockSpec((1, HIDDEN), lambda i: (0, 0)),
            pl.BlockSpec((1, HIDDEN), lambda i: (0, 0)),
        ],
        out_specs=pl.BlockSpec((_SB, 128), lambda i: (i, 0)),
        out_shape=jax.ShapeDtypeStruct((N_PAD // 128, 128), jnp.float32),
    )(p0, p1, isd, w2row, b1row)


def _tc3(q0, q1, isd, b2):
    def body(q0_ref, q1_ref, isd_ref, b2_ref, out_ref):
        out_ref[...] = (isd_ref[...] * (q0_ref[...] + q1_ref[...])
                        + b2_ref[0, 0])

    nb = N_PAD // 128
    return pl.pallas_call(
        body,
        in_specs=[
            pl.BlockSpec((nb, 128), lambda: (0, 0)),
            pl.BlockSpec((nb, 128), lambda: (0, 0)),
            pl.BlockSpec((nb, 128), lambda: (0, 0)),
            pl.BlockSpec((1, 1), lambda: (0, 0)),
        ],
        out_specs=pl.BlockSpec((nb, 128), lambda: (0, 0)),
        out_shape=jax.ShapeDtypeStruct((nb, 128), jnp.float32),
    )(q0, q1, isd, b2)


def kernel(x, edge_index, W1, b1, W2, b2):
    ei_p = jnp.pad(edge_index, ((0, 0), (0, E_PAD - N_EDGES)))
    col = lax.broadcasted_iota(jnp.int32, (2, E_PAD), 1)
    ei_p = jnp.where(col < N_EDGES, ei_p,
                     N_NODES + col % (N_PAD - N_NODES))
    src3 = ei_p[0].reshape(NW, NCH, C)
    dst3 = ei_p[1].reshape(NW, NCH, C)

    h = _tc1a(x, W1)
    dp = _sc_degree(dst3)
    d0 = dp[0].reshape(N_PAD // 128, 128)
    d1 = dp[1].reshape(N_PAD // 128, 128)
    hp, isd = _tc1b(h, d0, d1)
    pp = _sc_gather_scatter(hp, src3, dst3, HIDDEN, False, True)
    sp = _tc2(pp[0], pp[1], isd, W2.reshape(1, HIDDEN),
              b1.reshape(1, HIDDEN))
    qp = _sc_gather_scatter(sp.reshape(N_PAD), src3, dst3, 1, True, True)
    out = _tc3(qp[0].reshape(N_PAD // 128, 128),
               qp[1].reshape(N_PAD // 128, 128), isd, b2.reshape(1, 1))
    return out.reshape(-1)[:N_NODES]


# speedup vs baseline: 1.1345x; 1.0679x over previous
import functools

import jax
import jax.numpy as jnp
from jax import lax
from jax.experimental import pallas as pl
from jax.experimental.pallas import tpu as pltpu
from jax.experimental.pallas import tpu_sc as plsc

N_NODES = 10000
N_PAD = 10240
N_EDGES = 320000
D_FEAT = 128
HIDDEN = 64

NC = 2
NS = 16
NW = NC * NS
C = 128
NCH = 80
EW = NCH * C
E_PAD = NW * EW
ZR = N_PAD // NS
NBUF = 8
GA = 6
SLAG = 2


def _mesh():
    return plsc.VectorSubcoreMesh(core_axis_name="c", subcore_axis_name="s")


_SC_PARAMS = pltpu.CompilerParams(use_tc_tiling_on_sc=False)


def _sc_degree(dst3):
    @functools.partial(
        pl.kernel,
        mesh=_mesh(),
        out_type=jax.ShapeDtypeStruct((NC, N_PAD), jnp.float32),
        scratch_types=[
            pltpu.VMEM((NCH, C), jnp.int32),
            pltpu.VMEM((C,), jnp.float32),
            pltpu.VMEM((ZR,), jnp.float32),
            pltpu.VMEM_SHARED((N_PAD,), jnp.float32),
            pltpu.SemaphoreType.DMA,
        ],
        compiler_params=_SC_PARAMS,
    )
    def k(dst_h, out_h, dst_v, ones_v, zbuf, acc, sem):
        c = lax.axis_index("c")
        s = lax.axis_index("s")
        wid = s * NC + c
        pltpu.sync_copy(dst_h.at[wid], dst_v)
        for i in range(C // 16):
            ones_v[pl.ds(i * 16, 16)] = jnp.ones((16,), jnp.float32)

        def zfill(i, carry):
            zbuf[pl.ds(i * 16, 16)] = jnp.zeros((16,), jnp.float32)
            return carry

        lax.fori_loop(0, ZR // 16, zfill, 0)
        pltpu.sync_copy(zbuf, acc.at[pl.ds(s * ZR, ZR)])
        plsc.subcore_barrier()

        def fire(j, carry):
            pltpu.async_copy(ones_v, acc.at[dst_v.at[j]], sem, add=True)
            return carry

        lax.fori_loop(0, NCH, fire, 0)

        def drain(j, carry):
            pltpu.make_async_copy(ones_v, acc.at[dst_v.at[0]], sem).wait()
            return carry

        lax.fori_loop(0, NCH, drain, 0)
        plsc.subcore_barrier()
        pltpu.sync_copy(acc.at[pl.ds(s * ZR, ZR)], out_h.at[c, pl.ds(s * ZR, ZR)])

    return k(dst3)


def _sc_gather_scatter(table, src3, dst3, feat, stage_table, init_from_table):
    wide = feat > 1
    buf_t = pltpu.VMEM((NBUF, C, feat) if wide else (NBUF, C), jnp.float32)
    acc_t = pltpu.VMEM_SHARED((N_PAD, feat) if wide else (N_PAD,), jnp.float32)
    if wide:
        out_sh = [jax.ShapeDtypeStruct((N_PAD, feat), jnp.float32)
                  for _ in range(NC)]
    else:
        out_sh = jax.ShapeDtypeStruct((NC, N_PAD), jnp.float32)
    scratch = [
        pltpu.VMEM((NCH, C), jnp.int32),
        pltpu.VMEM((NCH, C), jnp.int32),
        buf_t,
        acc_t,
    ]
    if stage_table:
        scratch.append(pltpu.VMEM_SHARED((N_PAD, feat) if wide else (N_PAD,),
                                         jnp.float32))
    scratch += [pltpu.SemaphoreType.DMA] * (2 * NBUF)

    @functools.partial(
        pl.kernel,
        mesh=_mesh(),
        out_type=out_sh,
        scratch_types=scratch,
        compiler_params=_SC_PARAMS,
    )
    def k(tab_h, src_h, dst_h, *out_and_rest):
        if wide:
            outs = out_and_rest[:NC]
            src_v, dst_v, bufs = out_and_rest[NC:NC + 3]
            rest = out_and_rest[NC + 3:]
        else:
            out_h = out_and_rest[0]
            src_v, dst_v, bufs = out_and_rest[1:4]
            rest = out_and_rest[4:]
        if stage_table:
            tab = rest[1]
            acc = rest[0]
            sems = rest[2:]
        else:
            tab = tab_h
            acc = rest[0]
            sems = rest[1:]
        gsem = sems[:NBUF]
        ssem = sems[NBUF:]
        c = lax.axis_index("c")
        s = lax.axis_index("s")
        wid = s * NC + c
        pltpu.sync_copy(src_h.at[wid], src_v)
        pltpu.sync_copy(dst_h.at[wid], dst_v)
        if wide:
            def zfill(i, carry):
                for j in range(feat // 16):
                    bufs[0, i, pl.ds(j * 16, 16)] = jnp.zeros((16,),
                                                              jnp.float32)
                return carry
        else:
            def zfill(i, carry):
                bufs[0, pl.ds(i * 16, 16)] = jnp.zeros((16,), jnp.float32)
                return carry

        lax.fori_loop(0, C if wide else C // 16, zfill, 0)
        if init_from_table:
            @pl.when(c == 0)
            def _():
                pltpu.sync_copy(tab_h.at[pl.ds(s * ZR, ZR)],
                                acc.at[pl.ds(s * ZR, ZR)])

            @pl.when(c != 0)
            def _():
                for kk in range(ZR // C):
                    pltpu.sync_copy(bufs.at[0],
                                    acc.at[pl.ds(s * ZR + kk * C, C)])
        else:
            for kk in range(ZR // C):
                pltpu.sync_copy(bufs.at[0], acc.at[pl.ds(s * ZR + kk * C, C)])
        if stage_table:
            pltpu.sync_copy(tab_h.at[pl.ds(s * ZR, ZR)],
                            tab.at[pl.ds(s * ZR, ZR)])
        plsc.subcore_barrier()

        def gfire(j, b):
            pltpu.async_copy(tab.at[src_v.at[j]], bufs.at[b], gsem[b])

        def gwait(j, b):
            pltpu.make_async_copy(tab.at[src_v.at[j]], bufs.at[b],
                                  gsem[b]).wait()

        def sfire(j, b):
            pltpu.async_copy(bufs.at[b], acc.at[dst_v.at[j]], ssem[b],
                             add=True)

        def sdrain(j, b):
            pltpu.make_async_copy(bufs.at[b], acc.at[dst_v.at[j]],
                                  ssem[b]).wait()

        for j in range(GA):
            gfire(j, j)
        for b in range(NBUF):
            gwait(b, b)
            sfire(b, b)
            if b >= SLAG:
                sdrain(b - SLAG, (b - SLAG) % NBUF)
            gfire(b + GA, (b + GA) % NBUF)

        def body(t, carry):
            for b in range(NBUF):
                j = t * NBUF + b
                gwait(j, b)
                sfire(j, b)
                sdrain(j - SLAG, (b - SLAG) % NBUF)
                gfire(j + GA, (b + GA) % NBUF)
            return carry

        lax.fori_loop(1, (NCH // NBUF) - 1, body, 0)
        t = NCH // NBUF - 1
        for b in range(NBUF):
            j = t * NBUF + b
            gwait(j, b)
            sfire(j, b)
            sdrain(j - SLAG, (b - SLAG) % NBUF)
            if j + GA < NCH:
                gfire(j + GA, (b + GA) % NBUF)
        for b in range(NBUF - SLAG, NBUF):
            sdrain(t * NBUF + b, b)
        plsc.subcore_barrier()
        if wide:
            for cc in range(NC):
                @pl.when(c == cc)
                def _():
                    pltpu.sync_copy(acc.at[pl.ds(s * ZR, ZR)],
                                    outs[cc].at[pl.ds(s * ZR, ZR)])
        else:
            pltpu.sync_copy(acc.at[pl.ds(s * ZR, ZR)],
                            out_h.at[c, pl.ds(s * ZR, ZR)])

    return k(table, src3, dst3)


_RB = 2048
_SB = _RB // 128


def _tc1a(x_p, W1):
    def body(x_ref, w_ref, h_ref):
        h_ref[...] = jnp.dot(x_ref[...].astype(jnp.bfloat16),
                             w_ref[...].astype(jnp.bfloat16),
                             preferred_element_type=jnp.float32)

    grid = (N_PAD // _RB,)
    return pl.pallas_call(
        body,
        grid=grid,
        in_specs=[
            pl.BlockSpec((_RB, D_FEAT), lambda i: (i, 0)),
            pl.BlockSpec((D_FEAT, HIDDEN), lambda i: (0, 0)),
        ],
        out_specs=pl.BlockSpec((_RB, HIDDEN), lambda i: (i, 0)),
        out_shape=jax.ShapeDtypeStruct((N_PAD, HIDDEN), jnp.float32),
    )(x_p, W1)


def _tc1b(h, d0, d1):
    def body(h_ref, d0_ref, d1_ref, hp_ref, isd_ref):
        deg = d0_ref[...] + d1_ref[...] + 1.0
        isd = lax.rsqrt(deg)
        h3 = h_ref[...].reshape(_SB, 128, HIDDEN)
        hp_ref[...] = (h3 * isd.reshape(_SB, 128, 1)).reshape(_RB, HIDDEN)
        isd_ref[...] = isd

    grid = (N_PAD // _RB,)
    return pl.pallas_call(
        body,
        grid=grid,
        in_specs=[
            pl.BlockSpec((_RB, HIDDEN), lambda i: (i, 0)),
            pl.BlockSpec((_SB, 128), lambda i: (i, 0)),
            pl.BlockSpec((_SB, 128), lambda i: (i, 0)),
        ],
        out_specs=[
            pl.BlockSpec((_RB, HIDDEN), lambda i: (i, 0)),
            pl.BlockSpec((_SB, 128), lambda i: (i, 0)),
        ],
        out_shape=[
            jax.ShapeDtypeStruct((N_PAD, HIDDEN), jnp.float32),
            jax.ShapeDtypeStruct((N_PAD // 128, 128), jnp.float32),
        ],
    )(h, d0, d1)


def _tc2(p0, p1, isd, w2row, b1row):
    def body(p0_ref, p1_ref, isd_ref, w2_ref, b1_ref, sp_ref):
        isd3 = isd_ref[...].reshape(_SB, 128, 1)
        pre3 = ((p0_ref[...] + p1_ref[...]).reshape(_SB, 128, HIDDEN) * isd3
                + b1_ref[...].reshape(1, 1, HIDDEN))
        r3 = jnp.maximum(pre3, 0.0)
        r3b = r3.astype(jnp.bfloat16).astype(jnp.float32)
        w2b = (w2_ref[...].astype(jnp.bfloat16).astype(jnp.float32)
               .reshape(1, 1, HIDDEN))
        s3 = jnp.sum(r3b * w2b, axis=-1)
        sp_ref[...] = s3 * isd_ref[...]

    grid = (N_PAD // _RB,)
    return pl.pallas_call(
        body,
        grid=grid,
        in_specs=[
            pl.BlockSpec((_RB, HIDDEN), lambda i: (i, 0)),
            pl.BlockSpec((_RB, HIDDEN), lambda i: (i, 0)),
            pl.BlockSpec((_SB, 128), lambda i: (i, 0)),
            pl.BlockSpec((1, HIDDEN), lambda i: (0, 0)),
            pl.BlockSpec((1, HIDDEN), lambda i: (0, 0)),
        ],
        out_specs=pl.BlockSpec((_SB, 128), lambda i: (i, 0)),
        out_shape=jax.ShapeDtypeStruct((N_PAD // 128, 128), jnp.float32),
    )(p0, p1, isd, w2row, b1row)


def _tc3(q0, q1, isd, b2):
    def body(q0_ref, q1_ref, isd_ref, b2_ref, out_ref):
        out_ref[...] = (isd_ref[...] * (q0_ref[...] + q1_ref[...])
                        + b2_ref[0, 0])

    nb = N_PAD // 128
    return pl.pallas_call(
        body,
        in_specs=[
            pl.BlockSpec((nb, 128), lambda: (0, 0)),
            pl.BlockSpec((nb, 128), lambda: (0, 0)),
            pl.BlockSpec((nb, 128), lambda: (0, 0)),
            pl.BlockSpec((1, 1), lambda: (0, 0)),
        ],
        out_specs=pl.BlockSpec((nb, 128), lambda: (0, 0)),
        out_shape=jax.ShapeDtypeStruct((nb, 128), jnp.float32),
    )(q0, q1, isd, b2)


def kernel(x, edge_index, W1, b1, W2, b2):
    ei_p = jnp.pad(edge_index, ((0, 0), (0, E_PAD - N_EDGES)))
    col = lax.broadcasted_iota(jnp.int32, (2, E_PAD), 1)
    ei_p = jnp.where(col < N_EDGES, ei_p,
                     N_NODES + col % (N_PAD - N_NODES))
    src3 = ei_p[0].reshape(NW, NCH, C)
    dst3 = ei_p[1].reshape(NW, NCH, C)

    h = _tc1a(x, W1)
    dp = _sc_degree(dst3)
    d0 = dp[0].reshape(N_PAD // 128, 128)
    d1 = dp[1].reshape(N_PAD // 128, 128)
    hp, isd = _tc1b(h, d0, d1)
    pp = _sc_gather_scatter(hp, src3, dst3, HIDDEN, False, True)
    sp = _tc2(pp[0], pp[1], isd, W2.reshape(1, HIDDEN),
              b1.reshape(1, HIDDEN))
    qp = _sc_gather_scatter(sp.reshape(N_PAD), src3, dst3, 1, True, True)
    out = _tc3(qp[0].reshape(N_PAD // 128, 128),
               qp[1].reshape(N_PAD // 128, 128), isd, b2.reshape(1, 1))
    return out.reshape(-1)[:N_NODES]


# speedup vs baseline: 1.1625x; 1.0247x over previous
import functools

import jax
import jax.numpy as jnp
from jax import lax
from jax.experimental import pallas as pl
from jax.experimental.pallas import tpu as pltpu
from jax.experimental.pallas import tpu_sc as plsc

N_NODES = 10000
N_PAD = 10240
N_EDGES = 320000
D_FEAT = 128
HIDDEN = 64

NC = 2
NS = 16
NW = NC * NS
C = 128
NCH = 80
EW = NCH * C
E_PAD = NW * EW
ZR = N_PAD // NS
NBUF = 8
GA = 6
SLAG = 2


def _mesh():
    return plsc.VectorSubcoreMesh(core_axis_name="c", subcore_axis_name="s")


_SC_PARAMS = pltpu.CompilerParams(use_tc_tiling_on_sc=False)


def _sc_degree(dst3):
    @functools.partial(
        pl.kernel,
        mesh=_mesh(),
        out_type=[jax.ShapeDtypeStruct((N_PAD,), jnp.float32)
                  for _ in range(NC)],
        scratch_types=[
            pltpu.VMEM((NCH, C), jnp.int32),
            pltpu.VMEM((C,), jnp.float32),
            pltpu.VMEM((ZR,), jnp.float32),
            pltpu.VMEM_SHARED((N_PAD,), jnp.float32),
            pltpu.SemaphoreType.DMA,
        ],
        compiler_params=_SC_PARAMS,
    )
    def k(dst_h, out0_h, out1_h, dst_v, ones_v, zbuf, acc, sem):
        outs = (out0_h, out1_h)
        c = lax.axis_index("c")
        s = lax.axis_index("s")
        wid = s * NC + c
        pltpu.sync_copy(dst_h.at[wid], dst_v)
        for i in range(C // 16):
            ones_v[pl.ds(i * 16, 16)] = jnp.ones((16,), jnp.float32)

        def zfill(i, carry):
            zbuf[pl.ds(i * 16, 16)] = jnp.zeros((16,), jnp.float32)
            return carry

        lax.fori_loop(0, ZR // 16, zfill, 0)
        pltpu.sync_copy(zbuf, acc.at[pl.ds(s * ZR, ZR)])
        plsc.subcore_barrier()

        def fire(j, carry):
            pltpu.async_copy(ones_v, acc.at[dst_v.at[j]], sem, add=True)
            return carry

        lax.fori_loop(0, NCH, fire, 0)

        def drain(j, carry):
            pltpu.make_async_copy(ones_v, acc.at[dst_v.at[0]], sem).wait()
            return carry

        lax.fori_loop(0, NCH, drain, 0)
        plsc.subcore_barrier()
        for cc in range(NC):
            @pl.when(c == cc)
            def _():
                pltpu.sync_copy(acc.at[pl.ds(s * ZR, ZR)],
                                outs[cc].at[pl.ds(s * ZR, ZR)])

    return k(dst3)


def _sc_gather_scatter(table, src3, dst3, feat, stage_table, init_from_table):
    wide = feat > 1
    buf_t = pltpu.VMEM((NBUF, C, feat) if wide else (NBUF, C), jnp.float32)
    acc_t = pltpu.VMEM_SHARED((N_PAD, feat) if wide else (N_PAD,), jnp.float32)
    out_sh = [jax.ShapeDtypeStruct((N_PAD, feat) if wide else (N_PAD,),
                                   jnp.float32) for _ in range(NC)]
    scratch = [
        pltpu.VMEM((NCH, C), jnp.int32),
        pltpu.VMEM((NCH, C), jnp.int32),
        buf_t,
        acc_t,
    ]
    if stage_table:
        scratch.append(pltpu.VMEM_SHARED((N_PAD, feat) if wide else (N_PAD,),
                                         jnp.float32))
    scratch += [pltpu.SemaphoreType.DMA] * (2 * NBUF)

    @functools.partial(
        pl.kernel,
        mesh=_mesh(),
        out_type=out_sh,
        scratch_types=scratch,
        compiler_params=_SC_PARAMS,
    )
    def k(tab_h, src_h, dst_h, *out_and_rest):
        outs = out_and_rest[:NC]
        src_v, dst_v, bufs = out_and_rest[NC:NC + 3]
        rest = out_and_rest[NC + 3:]
        if stage_table:
            tab = rest[1]
            acc = rest[0]
            sems = rest[2:]
        else:
            tab = tab_h
            acc = rest[0]
            sems = rest[1:]
        gsem = sems[:NBUF]
        ssem = sems[NBUF:]
        c = lax.axis_index("c")
        s = lax.axis_index("s")
        wid = s * NC + c
        pltpu.sync_copy(src_h.at[wid], src_v)
        pltpu.sync_copy(dst_h.at[wid], dst_v)
        if wide:
            def zfill(i, carry):
                for j in range(feat // 16):
                    bufs[0, i, pl.ds(j * 16, 16)] = jnp.zeros((16,),
                                                              jnp.float32)
                return carry
        else:
            def zfill(i, carry):
                bufs[0, pl.ds(i * 16, 16)] = jnp.zeros((16,), jnp.float32)
                return carry

        lax.fori_loop(0, C if wide else C // 16, zfill, 0)
        if init_from_table:
            @pl.when(c == 0)
            def _():
                pltpu.sync_copy(tab_h.at[pl.ds(s * ZR, ZR)],
                                acc.at[pl.ds(s * ZR, ZR)])

            @pl.when(c != 0)
            def _():
                for kk in range(ZR // C):
                    pltpu.sync_copy(bufs.at[0],
                                    acc.at[pl.ds(s * ZR + kk * C, C)])
        else:
            for kk in range(ZR // C):
                pltpu.sync_copy(bufs.at[0], acc.at[pl.ds(s * ZR + kk * C, C)])
        if stage_table:
            pltpu.sync_copy(tab_h.at[pl.ds(s * ZR, ZR)],
                            tab.at[pl.ds(s * ZR, ZR)])
        plsc.subcore_barrier()

        def gfire(j, b):
            pltpu.async_copy(tab.at[src_v.at[j]], bufs.at[b], gsem[b])

        def gwait(j, b):
            pltpu.make_async_copy(tab.at[src_v.at[j]], bufs.at[b],
                                  gsem[b]).wait()

        def sfire(j, b):
            pltpu.async_copy(bufs.at[b], acc.at[dst_v.at[j]], ssem[b],
                             add=True)

        def sdrain(j, b):
            pltpu.make_async_copy(bufs.at[b], acc.at[dst_v.at[j]],
                                  ssem[b]).wait()

        for j in range(GA):
            gfire(j, j)
        for b in range(NBUF):
            gwait(b, b)
            sfire(b, b)
            if b >= SLAG:
                sdrain(b - SLAG, (b - SLAG) % NBUF)
            gfire(b + GA, (b + GA) % NBUF)

        def body(t, carry):
            for b in range(NBUF):
                j = t * NBUF + b
                gwait(j, b)
                sfire(j, b)
                sdrain(j - SLAG, (b - SLAG) % NBUF)
                gfire(j + GA, (b + GA) % NBUF)
            return carry

        lax.fori_loop(1, (NCH // NBUF) - 1, body, 0)
        t = NCH // NBUF - 1
        for b in range(NBUF):
            j = t * NBUF + b
            gwait(j, b)
            sfire(j, b)
            sdrain(j - SLAG, (b - SLAG) % NBUF)
            if j + GA < NCH:
                gfire(j + GA, (b + GA) % NBUF)
        for b in range(NBUF - SLAG, NBUF):
            sdrain(t * NBUF + b, b)
        plsc.subcore_barrier()
        for cc in range(NC):
            @pl.when(c == cc)
            def _():
                pltpu.sync_copy(acc.at[pl.ds(s * ZR, ZR)],
                                outs[cc].at[pl.ds(s * ZR, ZR)])

    return k(table, src3, dst3)


_RB = 2048
_SB = _RB // 128


def _tc1a(x_p, W1):
    def body(x_ref, w_ref, h_ref):
        h_ref[...] = jnp.dot(x_ref[...].astype(jnp.bfloat16),
                             w_ref[...].astype(jnp.bfloat16),
                             preferred_element_type=jnp.float32)

    grid = (N_PAD // _RB,)
    return pl.pallas_call(
        body,
        grid=grid,
        in_specs=[
            pl.BlockSpec((_RB, D_FEAT), lambda i: (i, 0)),
            pl.BlockSpec((D_FEAT, HIDDEN), lambda i: (0, 0)),
        ],
        out_specs=pl.BlockSpec((_RB, HIDDEN), lambda i: (i, 0)),
        out_shape=jax.ShapeDtypeStruct((N_PAD, HIDDEN), jnp.float32),
    )(x_p, W1)


def _tc1b(h, d0, d1):
    def body(h_ref, d0_ref, d1_ref, hp_ref, isd_ref):
        deg = d0_ref[...] + d1_ref[...] + 1.0
        isd = lax.rsqrt(deg)
        h3 = h_ref[...].reshape(_SB, 128, HIDDEN)
        hp_ref[...] = (h3 * isd.reshape(_SB, 128, 1)).reshape(_RB, HIDDEN)
        isd_ref[...] = isd

    grid = (N_PAD // _RB,)
    return pl.pallas_call(
        body,
        grid=grid,
        in_specs=[
            pl.BlockSpec((_RB, HIDDEN), lambda i: (i, 0)),
            pl.BlockSpec((_SB, 128), lambda i: (i, 0)),
            pl.BlockSpec((_SB, 128), lambda i: (i, 0)),
        ],
        out_specs=[
            pl.BlockSpec((_RB, HIDDEN), lambda i: (i, 0)),
            pl.BlockSpec((_SB, 128), lambda i: (i, 0)),
        ],
        out_shape=[
            jax.ShapeDtypeStruct((N_PAD, HIDDEN), jnp.float32),
            jax.ShapeDtypeStruct((N_PAD // 128, 128), jnp.float32),
        ],
    )(h, d0, d1)


def _tc2(p0, p1, isd, w2row, b1row):
    def body(p0_ref, p1_ref, isd_ref, w2_ref, b1_ref, sp_ref):
        isd3 = isd_ref[...].reshape(_SB, 128, 1)
        pre3 = ((p0_ref[...] + p1_ref[...]).reshape(_SB, 128, HIDDEN) * isd3
                + b1_ref[...].reshape(1, 1, HIDDEN))
        r3 = jnp.maximum(pre3, 0.0)
        r3b = r3.astype(jnp.bfloat16).astype(jnp.float32)
        w2b = (w2_ref[...].astype(jnp.bfloat16).astype(jnp.float32)
               .reshape(1, 1, HIDDEN))
        s3 = jnp.sum(r3b * w2b, axis=-1)
        sp_ref[...] = s3 * isd_ref[...]

    grid = (N_PAD // _RB,)
    return pl.pallas_call(
        body,
        grid=grid,
        in_specs=[
            pl.BlockSpec((_RB, HIDDEN), lambda i: (i, 0)),
            pl.BlockSpec((_RB, HIDDEN), lambda i: (i, 0)),
            pl.BlockSpec((_SB, 128), lambda i: (i, 0)),
            pl.BlockSpec((1, HIDDEN), lambda i: (0, 0)),
            pl.BlockSpec((1, HIDDEN), lambda i: (0, 0)),
        ],
        out_specs=pl.BlockSpec((_SB, 128), lambda i: (i, 0)),
        out_shape=jax.ShapeDtypeStruct((N_PAD // 128, 128), jnp.float32),
    )(p0, p1, isd, w2row, b1row)


def _tc3(q0, q1, isd, b2):
    def body(q0_ref, q1_ref, isd_ref, b2_ref, out_ref):
        out_ref[...] = (isd_ref[...] * (q0_ref[...] + q1_ref[...])
                        + b2_ref[0, 0])

    nb = N_PAD // 128
    return pl.pallas_call(
        body,
        in_specs=[
            pl.BlockSpec((nb, 128), lambda: (0, 0)),
            pl.BlockSpec((nb, 128), lambda: (0, 0)),
            pl.BlockSpec((nb, 128), lambda: (0, 0)),
            pl.BlockSpec((1, 1), lambda: (0, 0)),
        ],
        out_specs=pl.BlockSpec((nb, 128), lambda: (0, 0)),
        out_shape=jax.ShapeDtypeStruct((nb, 128), jnp.float32),
    )(q0, q1, isd, b2)


def kernel(x, edge_index, W1, b1, W2, b2):
    ei_p = jnp.pad(edge_index, ((0, 0), (0, E_PAD - N_EDGES)))
    col = lax.broadcasted_iota(jnp.int32, (2, E_PAD), 1)
    ei_p = jnp.where(col < N_EDGES, ei_p,
                     N_NODES + col % (N_PAD - N_NODES))
    src3 = ei_p[0].reshape(NW, NCH, C)
    dst3 = ei_p[1].reshape(NW, NCH, C)

    h = _tc1a(x, W1)
    dp0, dp1 = _sc_degree(dst3)
    d0 = dp0.reshape(N_PAD // 128, 128)
    d1 = dp1.reshape(N_PAD // 128, 128)
    hp, isd = _tc1b(h, d0, d1)
    pp = _sc_gather_scatter(hp, src3, dst3, HIDDEN, False, True)
    sp = _tc2(pp[0], pp[1], isd, W2.reshape(1, HIDDEN),
              b1.reshape(1, HIDDEN))
    q0, q1 = _sc_gather_scatter(sp.reshape(N_PAD), src3, dst3, 1, True, True)
    out = _tc3(q0.reshape(N_PAD // 128, 128),
               q1.reshape(N_PAD // 128, 128), isd, b2.reshape(1, 1))
    return out.reshape(-1)[:N_NODES]
